# Initial kernel scaffold; baseline (speedup 1.0000x reference)
import jax
import jax.numpy as jnp
from jax.experimental import pallas as pl


def kernel(voxel_features, voxel_indices):
    raise NotImplementedError("write your pallas kernel here")


# speedup vs baseline: 1.4622x; 1.4622x over previous
import functools

import jax
import jax.numpy as jnp
from jax import lax
from jax.experimental import pallas as pl
from jax.experimental.pallas import tpu as pltpu
from jax.experimental.pallas import tpu_sc as plsc

NB = 4
C = 128
D = 2
H = 200
W = 176
NV = 60000
NPOS = NB * D * H * W

NT = 32
RNG = NPOS // NT
SENT = 1 << 30
CH = 6000
NCH = NV // CH
LROWS = RNG // 128 + 2
PADR = NT * 128
HB = 8


def _sc_body(idx_hbm, vf_hbm, dense_hbm, valid_hbm,
             idxbuf, inv_v, vbuf, plist, ilist, rowbuf, gsem, ssem):
    cid = lax.axis_index("c")
    sid = lax.axis_index("s")
    wid = sid * 2 + cid
    base = wid * RNG
    iota = lax.broadcasted_iota(jnp.int32, (16,), 0)

    sent_v = jnp.full((16,), SENT, jnp.int32)

    def init_b(j, _):
        inv_v[pl.ds(j * 16, 16)] = sent_v
        return 0

    lax.fori_loop(0, RNG // 16, init_b, 0)

    def chunk_b(ci, _):
        pltpu.sync_copy(idx_hbm.at[pl.ds(ci * CH, CH)], idxbuf)

        def win_b(j, _):
            v = idxbuf[pl.ds(j * 16, 16)]
            m = (v >= base) & (v < base + RNG)
            li = jnp.where(m, v - base, 0)
            ids = ci * CH + j * 16 + iota
            plsc.store_scatter(inv_v, [li], ids, mask=m)
            g = plsc.load_gather(inv_v, [li], mask=m)
            m2 = m & (g < ids)

            def cond(m2_):
                return jnp.max(plsc.all_reduce_population_count(m2_)) > 0

            def fix(m2_):
                plsc.store_scatter(inv_v, [li], ids, mask=m2_)
                g2 = plsc.load_gather(inv_v, [li], mask=m)
                return m & (g2 < ids)

            lax.while_loop(cond, fix, m2)
            return 0

        lax.fori_loop(0, CH // 16, win_b, 0)
        return 0

    lax.fori_loop(0, NCH, chunk_b, 0)

    def comp_b(j, cnt):
        g = inv_v[pl.ds(j * 16, 16)]
        m = g < NV
        vbuf[pl.ds(j * 16, 16)] = jnp.where(m, 1, 0)
        c = plsc.cumsum(m.astype(jnp.int32))
        addr = cnt + c - 1
        row = lax.shift_right_logical(addr, 7)
        col = addr & 127
        pos = base + j * 16 + iota
        plsc.store_scatter(plist, [row, col], pos, mask=m)
        plsc.store_scatter(ilist, [row, col], g, mask=m)
        return cnt + plsc.all_reduce_population_count(m)

    cnt = lax.fori_loop(0, RNG // 16, comp_b, jnp.zeros((16,), jnp.int32))
    cnt_s = jnp.max(cnt)

    def pad_b(j, _):
        addr = cnt_s + j * 16 + iota
        row = lax.shift_right_logical(addr, 7)
        col = addr & 127
        pos = NPOS + wid * 128 + j * 16 + iota
        plsc.store_scatter(plist, [row, col], pos)
        plsc.store_scatter(ilist, [row, col], jnp.zeros((16,), jnp.int32))
        return 0

    lax.fori_loop(0, 8, pad_b, 0)
    trips = (cnt_s + 127) // 128

    def trip_b(t, _):
        pltpu.async_copy(vf_hbm.at[ilist.at[t]], rowbuf, gsem).wait()
        pltpu.async_copy(rowbuf, dense_hbm.at[plist.at[t]], ssem).wait()
        return 0

    lax.fori_loop(0, trips, trip_b, 0)

    pltpu.sync_copy(vbuf, valid_hbm.at[pl.ds(base, RNG)])


_sc_scatter = functools.partial(
    pl.kernel,
    out_type=(
        jax.ShapeDtypeStruct((NPOS + PADR, C), jnp.float32),
        jax.ShapeDtypeStruct((NPOS,), jnp.int32),
    ),
    mesh=plsc.VectorSubcoreMesh(core_axis_name="c", subcore_axis_name="s"),
    compiler_params=pltpu.CompilerParams(needs_layout_passes=False),
    scratch_types=(
        pltpu.VMEM((CH,), jnp.int32),
        pltpu.VMEM((RNG,), jnp.int32),
        pltpu.VMEM((RNG,), jnp.int32),
        pltpu.VMEM((LROWS, 128), jnp.int32),
        pltpu.VMEM((LROWS, 128), jnp.int32),
        pltpu.VMEM((128, C), jnp.float32),
        pltpu.SemaphoreType.DMA,
        pltpu.SemaphoreType.DMA,
    ),
)(_sc_body)


def _tr_body(x_ref, v_ref, o_ref):
    for h in range(HB):
        x = x_ref[pl.ds(h * W, W), :]
        y = x.T
        m = v_ref[0, 0, h, :]
        y = jnp.where((m != 0)[None, :], y, 0.0)
        o_ref[0, :, 0, h, :] = y


def _transpose(dense, valid4):
    return pl.pallas_call(
        _tr_body,
        grid=(NB, D, H // HB),
        in_specs=[
            pl.BlockSpec((HB * W, C),
                         lambda n, d, hb: (n * (D * H // HB) + d * (H // HB) + hb, 0)),
            pl.BlockSpec((1, 1, HB, W), lambda n, d, hb: (n, d, hb, 0)),
        ],
        out_specs=pl.BlockSpec((1, C, 1, HB, W), lambda n, d, hb: (n, 0, d, hb, 0)),
        out_shape=jax.ShapeDtypeStruct((NB, C, D, H, W), jnp.float32),
    )(dense, valid4)


@jax.jit
def kernel(voxel_features, voxel_indices):
    dense, valid = _sc_scatter(voxel_indices, voxel_features)
    out5 = _transpose(dense, valid.reshape(NB, D, H, W))
    return out5.reshape(NB, C * D, H, W)


# speedup vs baseline: 2.8795x; 1.9694x over previous
import functools

import jax
import jax.numpy as jnp
from jax import lax
from jax.experimental import pallas as pl
from jax.experimental.pallas import tpu as pltpu
from jax.experimental.pallas import tpu_sc as plsc

NB = 4
C = 128
D = 2
H = 200
W = 176
NV = 60000
NPOS = NB * D * H * W

NT = 32
RNG = NPOS // NT
SENT = 1 << 30
CH = 6000
NCH = NV // CH
RB = 128
NBUF = 4
LROWS = RNG // RB + 2
PADR = NT * 128
HB = 8


def _sc_body(idx_hbm, vf_hbm, dense_hbm, valid_hbm,
             idxbuf, inv_v, vbuf, plist, ilist, rowbuf,
             gsem, ssem, csem):
    cid = lax.axis_index("c")
    sid = lax.axis_index("s")
    wid = sid * 2 + cid
    base = wid * RNG
    iota = lax.broadcasted_iota(jnp.int32, (16,), 0)

    sent_v = jnp.full((16,), SENT, jnp.int32)

    def init_b(j, _):
        for u in range(5):
            inv_v[pl.ds((j * 5 + u) * 16, 16)] = sent_v
        return 0

    with jax.named_scope("p1_init"):
        lax.fori_loop(0, RNG // 80, init_b, 0)

    rng_u = jnp.uint32(RNG)

    def _issue_chunk(ci):
        pltpu.async_copy(idx_hbm.at[pl.ds(ci * CH, CH)],
                         idxbuf.at[pl.ds(lax.rem(ci, 2) * CH, CH)], csem)

    def chunk_b(ci, acc):
        @pl.when(ci + 1 < NCH)
        def _prefetch():
            _issue_chunk(ci + 1)

        pltpu.make_async_copy(idx_hbm.at[pl.ds(0, CH)],
                              idxbuf.at[pl.ds(0, CH)], csem).wait()
        boff = lax.rem(ci, 2) * CH

        def win_b(j, acc):
            for u in range(5):
                jj = j * 5 + u
                v = idxbuf[pl.ds(boff + jj * 16, 16)]
                li = v - base
                m = plsc.bitcast(li, jnp.uint32) < rng_u
                ids = ci * CH + jj * 16 + iota
                plsc.store_scatter(inv_v, [li], ids, mask=m)
                g = plsc.load_gather(inv_v, [li], mask=m)
                acc = acc | (m & (g < ids))
            return acc

        return lax.fori_loop(0, CH // 80, win_b, acc)

    with jax.named_scope("p2_scan"):
        _issue_chunk(0)
        acc = lax.fori_loop(0, NCH, chunk_b, jnp.zeros((16,), jnp.bool_))
    dirty = jnp.max(plsc.all_reduce_population_count(acc)) > 0

    @pl.when(dirty)
    def _fixup():
        def fchunk_b(ci, _):
            pltpu.sync_copy(idx_hbm.at[pl.ds(ci * CH, CH)],
                            idxbuf.at[pl.ds(0, CH)])

            def win2(j, _):
                v = idxbuf[pl.ds(j * 16, 16)]
                li = v - base
                m = plsc.bitcast(li, jnp.uint32) < rng_u
                ids = ci * CH + j * 16 + iota
                g = plsc.load_gather(inv_v, [li], mask=m)
                m2 = m & (g < ids)

                def cond(mm):
                    return jnp.max(plsc.all_reduce_population_count(mm)) > 0

                def fix(mm):
                    plsc.store_scatter(inv_v, [li], ids, mask=mm)
                    g2 = plsc.load_gather(inv_v, [li], mask=m)
                    return m & (g2 < ids)

                lax.while_loop(cond, fix, m2)
                return 0

            lax.fori_loop(0, CH // 16, win2, 0)
            return 0

        lax.fori_loop(0, NCH, fchunk_b, 0)

    def comp_b(j, cnt):
        for u in range(5):
            jj = j * 5 + u
            g = inv_v[pl.ds(jj * 16, 16)]
            m = g < NV
            vbuf[pl.ds(jj * 16, 16)] = jnp.where(m, 1, 0)
            c = plsc.cumsum(m.astype(jnp.int32))
            addr = cnt + c - 1
            row = lax.shift_right_logical(addr, 7)
            col = addr & 127
            pos = base + jj * 16 + iota
            plsc.store_scatter(plist, [row, col], pos, mask=m)
            plsc.store_scatter(ilist, [row, col], g, mask=m)
            cnt = cnt + plsc.all_reduce_population_count(m)
        return cnt

    with jax.named_scope("p3_compress"):
        cnt = lax.fori_loop(0, RNG // 80, comp_b, jnp.zeros((16,), jnp.int32))
    cnt_s = jnp.max(cnt)

    def pad_b(j, _):
        addr = cnt_s + j * 16 + iota
        row = lax.shift_right_logical(addr, 7)
        col = addr & 127
        pos = NPOS + wid * 128 + j * 16 + iota
        plsc.store_scatter(plist, [row, col], pos)
        plsc.store_scatter(ilist, [row, col], jnp.zeros((16,), jnp.int32))
        return 0

    lax.fori_loop(0, RB // 16, pad_b, 0)
    trips = (cnt_s + RB - 1) // RB

    def _issue_gather(t):
        pltpu.async_copy(vf_hbm.at[ilist.at[t]],
                         rowbuf.at[lax.rem(t, NBUF)], gsem)

    def _drain(sem):
        pltpu.make_async_copy(vf_hbm.at[ilist.at[0]], rowbuf.at[0],
                              sem).wait()

    def prol_b(t, _):
        _issue_gather(t)
        return 0

    lax.fori_loop(0, jnp.minimum(trips, 2), prol_b, 0)

    def trip_b(t, _):
        @pl.when(t >= 2)
        def _drain_old_scatter():
            _drain(ssem)

        @pl.when(t + 2 < trips)
        def _prefetch():
            _issue_gather(t + 2)

        _drain(gsem)
        pltpu.async_copy(rowbuf.at[lax.rem(t, NBUF)],
                         dense_hbm.at[plist.at[t]], ssem)
        return 0

    with jax.named_scope("p4_streams"):
        lax.fori_loop(0, trips, trip_b, 0)

    def epi_b(k, _):
        _drain(ssem)
        return 0

    lax.fori_loop(0, jnp.minimum(trips, 2), epi_b, 0)

    with jax.named_scope("p5_valid"):
        pltpu.sync_copy(vbuf, valid_hbm.at[pl.ds(base, RNG)])


_sc_scatter = functools.partial(
    pl.kernel,
    out_type=(
        jax.ShapeDtypeStruct((NPOS + PADR, C), jnp.float32),
        jax.ShapeDtypeStruct((NPOS,), jnp.int32),
    ),
    mesh=plsc.VectorSubcoreMesh(core_axis_name="c", subcore_axis_name="s"),
    compiler_params=pltpu.CompilerParams(needs_layout_passes=False,
                                         use_tc_tiling_on_sc=True),
    scratch_types=(
        pltpu.VMEM((2 * CH,), jnp.int32),
        pltpu.VMEM((RNG,), jnp.int32),
        pltpu.VMEM((RNG,), jnp.int32),
        pltpu.VMEM((LROWS, RB), jnp.int32),
        pltpu.VMEM((LROWS, RB), jnp.int32),
        pltpu.VMEM((NBUF, RB, C), jnp.float32),
        pltpu.SemaphoreType.DMA,
        pltpu.SemaphoreType.DMA,
        pltpu.SemaphoreType.DMA,
    ),
)(_sc_body)


def _il_body(x0_ref, x1_ref, v0_ref, v1_ref, o_ref):
    x0 = x0_ref[...]
    x1 = x1_ref[...]
    vm0 = (v0_ref[...].reshape(HB * W) != 0).astype(jnp.float32)[:, None]
    vm1 = (v1_ref[...].reshape(HB * W) != 0).astype(jnp.float32)[:, None]
    row = lax.broadcasted_iota(jnp.int32, (C, 2 * C), 0)
    col = lax.broadcasted_iota(jnp.int32, (C, 2 * C), 1)
    e0 = (col == 2 * row).astype(jnp.bfloat16)
    e1 = (col == 2 * row + 1).astype(jnp.bfloat16)

    def sel(x, e):
        hi = x.astype(jnp.bfloat16)
        lo = (x - hi.astype(jnp.float32)).astype(jnp.bfloat16)
        return (jnp.dot(hi, e, preferred_element_type=jnp.float32)
                + jnp.dot(lo, e, preferred_element_type=jnp.float32))

    o_ref[...] = sel(x0 * vm0, e0) + sel(x1 * vm1, e1)


def _interleave(dense, valid3d):
    nhb = H // HB
    return pl.pallas_call(
        _il_body,
        grid=(NB, nhb),
        in_specs=[
            pl.BlockSpec((HB * W, C), lambda n, hb: (2 * n * nhb + hb, 0)),
            pl.BlockSpec((HB * W, C),
                         lambda n, hb: ((2 * n + 1) * nhb + hb, 0)),
            pl.BlockSpec((1, HB * W // 128, 128),
                         lambda n, hb: (2 * n * nhb + hb, 0, 0)),
            pl.BlockSpec((1, HB * W // 128, 128),
                         lambda n, hb: ((2 * n + 1) * nhb + hb, 0, 0)),
        ],
        out_specs=pl.BlockSpec((HB * W, 2 * C), lambda n, hb: (n * nhb + hb, 0)),
        out_shape=jax.ShapeDtypeStruct((NB * H * W, 2 * C), jnp.float32),
    )(dense, dense, valid3d, valid3d)


@jax.jit
def kernel(voxel_features, voxel_indices):
    dense, valid = _sc_scatter(voxel_indices, voxel_features)
    phys = _interleave(
        dense, valid.reshape(D * NB * H // HB, HB * W // 128, 128))
    return jnp.transpose(phys.reshape(NB, H, W, D * C), (0, 3, 1, 2))
